# initial kernel scaffold (unmeasured)
import jax
import jax.numpy as jnp
from jax import lax
from jax.experimental import pallas as pl
from jax.experimental.pallas import tpu as pltpu

N_DEV = 8
B = 2
SQ = 512
SC = 512
H_LOC = 8
DH = 64
D_MODEL = 768


def kernel(x, Wq, K_ext, V_ext, Wo):
    def body(x_ref, wq_ref, k_ref, v_ref, wo_ref, out_ref,
             k_all, v_all, comm, a2a_send, a2a_recv, ar_send, ar_recv):
        me = lax.axis_index("i")

        barrier = pltpu.get_barrier_semaphore()
        for d in range(1, N_DEV):
            pl.semaphore_signal(
                barrier, inc=1,
                device_id=((me + d) % N_DEV,),
                device_id_type=pl.DeviceIdType.MESH,
            )
        pl.semaphore_wait(barrier, N_DEV - 1)

        k_all[0] = k_ref[:, :, pl.ds(me * H_LOC, H_LOC), :]
        v_all[0] = v_ref[:, :, pl.ds(me * H_LOC, H_LOC), :]

        rdmas = []
        for d in range(1, N_DEV):
            tgt = (me + d) % N_DEV
            for kv, (ref, all_ref) in enumerate(((k_ref, k_all), (v_ref, v_all))):
                r = pltpu.make_async_remote_copy(
                    src_ref=ref.at[:, :, pl.ds(tgt * H_LOC, H_LOC), :],
                    dst_ref=all_ref.at[d],
                    send_sem=a2a_send.at[kv, d - 1],
                    recv_sem=a2a_recv.at[kv, d - 1],
                    device_id=(tgt,),
                    device_id_type=pl.DeviceIdType.MESH,
                )
                r.start()
                rdmas.append(r)
        for r in rdmas:
            r.wait()

        qi = lax.broadcasted_iota(jnp.int32, (SQ, SC), 0)
        kj = lax.broadcasted_iota(jnp.int32, (SQ, SC), 1)

        for b in range(B):
            q_b = jnp.dot(x_ref[b], wq_ref[:, :],
                          preferred_element_type=jnp.float32) * 0.125
            ctx_cols = []
            for h in range(H_LOC):
                q_bh = q_b[:, h * DH:(h + 1) * DH]

                def chunk_step(j, carry, q_bh=q_bh, b=b, h=h):
                    num, den = carry
                    c = (me - j) % N_DEV
                    k_c = k_all[j, b, :, h, :]
                    v_c = v_all[j, b, :, h, :]
                    s = lax.dot_general(
                        q_bh, k_c, (((1,), (1,)), ((), ())),
                        preferred_element_type=jnp.float32,
                    )
                    kjg = kj + c * SC
                    m = (jnp.abs(qi - kjg) <= 128) | (kjg < 32) | (qi < 32)
                    s = jnp.where(m, s, -1e9)
                    w = jnp.exp(s)
                    den = den + jnp.sum(w, axis=1, keepdims=True)
                    num = num + jnp.dot(w, v_c,
                                        preferred_element_type=jnp.float32)
                    return num, den

                num, den = lax.fori_loop(
                    0, N_DEV, chunk_step,
                    (jnp.zeros((SQ, DH), jnp.float32),
                     jnp.zeros((SQ, 1), jnp.float32)),
                )
                ctx_cols.append(num / den)
            ctx_b = jnp.concatenate(ctx_cols, axis=1)
            out_ref[b] = jnp.dot(ctx_b, wo_ref[:, :],
                                 preferred_element_type=jnp.float32)

        right = (me + 1) % N_DEV
        for hh in range(N_DEV - 1):
            src = out_ref if hh == 0 else comm.at[hh - 1]
            r = pltpu.make_async_remote_copy(
                src_ref=src,
                dst_ref=comm.at[hh],
                send_sem=ar_send.at[hh],
                recv_sem=ar_recv.at[hh],
                device_id=(right,),
                device_id_type=pl.DeviceIdType.MESH,
            )
            r.start()
            r.wait()
            out_ref[:, :, :] = out_ref[:, :, :] + comm[hh]

    return pl.pallas_call(
        body,
        out_shape=jax.ShapeDtypeStruct((B, SQ, D_MODEL), jnp.float32),
        in_specs=[pl.BlockSpec(memory_space=pltpu.VMEM)] * 5,
        out_specs=pl.BlockSpec(memory_space=pltpu.VMEM),
        scratch_shapes=[
            pltpu.VMEM((N_DEV, B, SC, H_LOC, DH), jnp.float32),
            pltpu.VMEM((N_DEV, B, SC, H_LOC, DH), jnp.float32),
            pltpu.VMEM((N_DEV - 1, B, SQ, D_MODEL), jnp.float32),
            pltpu.SemaphoreType.DMA((2, N_DEV - 1)),
            pltpu.SemaphoreType.DMA((2, N_DEV - 1)),
            pltpu.SemaphoreType.DMA((N_DEV - 1,)),
            pltpu.SemaphoreType.DMA((N_DEV - 1,)),
        ],
        compiler_params=pltpu.CompilerParams(collective_id=0),
    )(x, Wq, K_ext, V_ext, Wo)


# baseline (device time: 481982 ns/iter reference)
import jax
import jax.numpy as jnp
from jax import lax
from jax.experimental import pallas as pl
from jax.experimental.pallas import tpu as pltpu

N_DEV = 8
B = 2
SQ = 512
SC = 512
H_LOC = 8
DH = 64
HD = H_LOC * DH
D_MODEL = 768
CH = SQ // N_DEV


def kernel(x, Wq, K_ext, V_ext, Wo):
    K2 = K_ext.reshape(B, SC, N_DEV * HD)
    V2 = V_ext.reshape(B, SC, N_DEV * HD)

    def body(x_ref, wq_ref, k2_ref, v2_ref, wo_ref, out_ref,
             k_all, v_all, seed, rs_recv, ag_recv,
             local_sems, a2a_send, a2a_recv, rs_ss, rs_rs, ag_ss, ag_rs):
        me = lax.axis_index("i")
        right = (me + 1) % N_DEV

        barrier = pltpu.get_barrier_semaphore()
        for d in range(1, N_DEV):
            pl.semaphore_signal(
                barrier, inc=1,
                device_id=((me + d) % N_DEV,),
                device_id_type=pl.DeviceIdType.MESH,
            )
        pl.semaphore_wait(barrier, N_DEV - 1)

        cp_k = pltpu.make_async_copy(
            k2_ref.at[:, :, pl.ds(me * HD, HD)], k_all.at[0], local_sems.at[0])
        cp_v = pltpu.make_async_copy(
            v2_ref.at[:, :, pl.ds(me * HD, HD)], v_all.at[0], local_sems.at[1])
        cp_k.start()
        cp_v.start()

        rdmas = []
        for d in range(1, N_DEV):
            tgt = (me + d) % N_DEV
            for kv, (ref, all_ref) in enumerate(((k2_ref, k_all), (v2_ref, v_all))):
                r = pltpu.make_async_remote_copy(
                    src_ref=ref.at[:, :, pl.ds(tgt * HD, HD)],
                    dst_ref=all_ref.at[d],
                    send_sem=a2a_send.at[kv, d - 1],
                    recv_sem=a2a_recv.at[kv, d - 1],
                    device_id=(tgt,),
                    device_id_type=pl.DeviceIdType.MESH,
                )
                r.start()
                rdmas.append(r)
        cp_k.wait()
        cp_v.wait()
        for r in rdmas:
            r.wait()

        qi = lax.broadcasted_iota(jnp.int32, (SQ, SC), 0)
        kj = lax.broadcasted_iota(jnp.int32, (SQ, SC), 1)

        for b in range(B):
            q_b = jnp.dot(x_ref[b], wq_ref[:, :],
                          preferred_element_type=jnp.float32) * 0.125
            ctx_cols = []
            for h in range(H_LOC):
                q_bh = q_b[:, h * DH:(h + 1) * DH]

                def chunk_step(j, carry, q_bh=q_bh, b=b, h=h):
                    num, den = carry
                    c = (me - j) % N_DEV
                    k_c = k_all[j, b, :, h * DH:(h + 1) * DH]
                    v_c = v_all[j, b, :, h * DH:(h + 1) * DH]
                    s = lax.dot_general(
                        q_bh, k_c, (((1,), (1,)), ((), ())),
                        preferred_element_type=jnp.float32,
                    )
                    kjg = kj + c * SC
                    m = (jnp.abs(qi - kjg) <= 128) | (kjg < 32) | (qi < 32)
                    s = jnp.where(m, s, -1e9)
                    w = jnp.exp(s)
                    den = den + jnp.sum(w, axis=1, keepdims=True)
                    num = num + jnp.dot(w, v_c,
                                        preferred_element_type=jnp.float32)
                    return num, den

                num, den = lax.fori_loop(
                    0, N_DEV, chunk_step,
                    (jnp.zeros((SQ, DH), jnp.float32),
                     jnp.zeros((SQ, 1), jnp.float32)),
                )
                ctx_cols.append(num / den)
            ctx_b = jnp.concatenate(ctx_cols, axis=1)
            out_ref[b] = jnp.dot(ctx_b, wo_ref[:, :],
                                 preferred_element_type=jnp.float32)

        def out_chunk(r):
            return out_ref[:, pl.ds(r * CH, CH), :]

        seed[...] = out_chunk(me)
        for s in range(N_DEV - 1):
            src = seed if s == 0 else rs_recv.at[s - 1]
            r = pltpu.make_async_remote_copy(
                src_ref=src, dst_ref=rs_recv.at[s],
                send_sem=rs_ss.at[s], recv_sem=rs_rs.at[s],
                device_id=(right,), device_id_type=pl.DeviceIdType.MESH,
            )
            r.start()
            r.wait()
            rc = (me - 1 - s) % N_DEV
            rs_recv[s] = rs_recv[s] + out_chunk(rc)

        o = (me + 1) % N_DEV
        out_ref[:, pl.ds(o * CH, CH), :] = rs_recv[N_DEV - 2]
        for s in range(N_DEV - 1):
            src = rs_recv.at[N_DEV - 2] if s == 0 else ag_recv.at[s - 1]
            r = pltpu.make_async_remote_copy(
                src_ref=src, dst_ref=ag_recv.at[s],
                send_sem=ag_ss.at[s], recv_sem=ag_rs.at[s],
                device_id=(right,), device_id_type=pl.DeviceIdType.MESH,
            )
            r.start()
            r.wait()
            rc = (o - 1 - s) % N_DEV
            out_ref[:, pl.ds(rc * CH, CH), :] = ag_recv[s]

    return pl.pallas_call(
        body,
        out_shape=jax.ShapeDtypeStruct((B, SQ, D_MODEL), jnp.float32),
        in_specs=[
            pl.BlockSpec(memory_space=pltpu.VMEM),
            pl.BlockSpec(memory_space=pltpu.VMEM),
            pl.BlockSpec(memory_space=pltpu.MemorySpace.HBM),
            pl.BlockSpec(memory_space=pltpu.MemorySpace.HBM),
            pl.BlockSpec(memory_space=pltpu.VMEM),
        ],
        out_specs=pl.BlockSpec(memory_space=pltpu.VMEM),
        scratch_shapes=[
            pltpu.VMEM((N_DEV, B, SC, HD), jnp.float32),
            pltpu.VMEM((N_DEV, B, SC, HD), jnp.float32),
            pltpu.VMEM((B, CH, D_MODEL), jnp.float32),
            pltpu.VMEM((N_DEV - 1, B, CH, D_MODEL), jnp.float32),
            pltpu.VMEM((N_DEV - 1, B, CH, D_MODEL), jnp.float32),
            pltpu.SemaphoreType.DMA((2,)),
            pltpu.SemaphoreType.DMA((2, N_DEV - 1)),
            pltpu.SemaphoreType.DMA((2, N_DEV - 1)),
            pltpu.SemaphoreType.DMA((N_DEV - 1,)),
            pltpu.SemaphoreType.DMA((N_DEV - 1,)),
            pltpu.SemaphoreType.DMA((N_DEV - 1,)),
            pltpu.SemaphoreType.DMA((N_DEV - 1,)),
        ],
        compiler_params=pltpu.CompilerParams(
            collective_id=0, vmem_limit_bytes=60 * 1024 * 1024),
    )(x, Wq, K2, V2, Wo)
